# baseline (speedup 1.0000x reference)
import functools

import numpy as np
import jax
import jax.numpy as jnp
from jax.experimental import pallas as pl
from jax.experimental.pallas import tpu as pltpu

_B, _C, _IMG, _P = 32, 3, 224, 16
_HID = 128
_G = _IMG // _P
_NP = _G * _G
_N = _B * _NP
_ND = _C * _P * _P
_APAD = 256
_NCOPY = _C * _P


@functools.lru_cache(maxsize=1)
def _static_graph():
    r, c = np.triu_indices(_NP, k=1)
    e = np.stack([r.astype(np.int64), c.astype(np.int64)])
    e = np.tile(e[None], (_B, 1, 1)).reshape(-1, 2).T
    row, col = e[0], e[1]
    deg = np.zeros((_N,), np.float64)
    np.add.at(deg, col, 1.0)
    deg += 1.0
    dinv = deg ** -0.5
    A = np.zeros((_APAD, _APAD), np.float64)
    np.add.at(A, (col, row), dinv[row] * dinv[col])
    idx = np.arange(_APAD)
    A[idx, idx] += dinv[:_APAD] ** 2
    Sp = np.zeros((_B, _N), np.float64)
    Sp[0, :_APAD] = A[:_NP, :].sum(axis=0) / _NP
    for g in range(1, _B):
        Sp[g, g * _NP:(g + 1) * _NP] = 1.0 / _NP
    return A.astype(np.float32), Sp.astype(np.float32)


def _fused_body(x_ref, wemb_ref, bemb_ref, w1_ref, b1_ref, w2_ref, b2_ref,
                w3_ref, b3_ref, a_ref, sp_ref, out_ref, xh_ref, t_ref):
    g = pl.program_id(0)
    v = x_ref[...].reshape(_N, _P)

    @pl.when(g == 0)
    def _init():
        xh_ref[...] = jnp.broadcast_to(bemb_ref[...], (_N, _HID))

    xh_ref[...] += jnp.dot(v, wemb_ref[0],
                           preferred_element_type=jnp.float32)

    @pl.when(g == _NCOPY - 1)
    def _tail():
        h = xh_ref[...]
        a = a_ref[...]
        for w_ref, b_ref in ((w1_ref, b1_ref), (w2_ref, b2_ref)):
            t = jnp.dot(h, w_ref[...], preferred_element_type=jnp.float32)
            t_ref[...] = t
            t_ref[0:_APAD, :] = jnp.dot(a, t[0:_APAD, :],
                                        preferred_element_type=jnp.float32)
            h = jnp.maximum(t_ref[...] + b_ref[...], 0.0)
        p = jnp.dot(sp_ref[...], h, preferred_element_type=jnp.float32)
        out_ref[...] = (jnp.dot(p, w3_ref[...],
                                preferred_element_type=jnp.float32)
                        + b3_ref[...])


def kernel(x, W_embed, b_embed, W1, b1, W2, b2, W3, b3):
    A, Sp = _static_graph()
    x6 = x.reshape(_B, _C, _G, _P, _G, _P)
    wemb3 = W_embed.T.reshape(_NCOPY, _P, _HID)
    full = lambda shape: pl.BlockSpec(shape, lambda g: (0,) * len(shape))
    return pl.pallas_call(
        _fused_body,
        grid=(_NCOPY,),
        in_specs=[
            pl.BlockSpec((_B, 1, _G, 1, _G, _P),
                         lambda g: (0, g // _P, 0, g % _P, 0, 0)),
            pl.BlockSpec((1, _P, _HID), lambda g: (g, 0, 0)),
            full((1, _HID)),
            full((_HID, _HID)),
            full((1, _HID)),
            full((_HID, _HID)),
            full((1, _HID)),
            full((_HID, _HID)),
            full((1, _HID)),
            full((_APAD, _APAD)),
            full((_B, _N)),
        ],
        out_specs=pl.BlockSpec((_B, _HID), lambda g: (0, 0)),
        out_shape=jax.ShapeDtypeStruct((_B, _HID), jnp.float32),
        scratch_shapes=[
            pltpu.MemorySpace.VMEM((_N, _HID), jnp.float32),
            pltpu.MemorySpace.VMEM((_N, _HID), jnp.float32),
        ],
    )(x6, wemb3, b_embed.reshape(1, -1), W1.T, b1.reshape(1, -1),
      W2.T, b2.reshape(1, -1), W3.T, b3.reshape(1, -1),
      jnp.asarray(A), jnp.asarray(Sp))


# speedup vs baseline: 1.0823x; 1.0823x over previous
import functools

import numpy as np
import jax
import jax.numpy as jnp
from jax.experimental import pallas as pl
from jax.experimental.pallas import tpu as pltpu

_B, _C, _IMG, _P = 32, 3, 224, 16
_HID = 128
_G = _IMG // _P
_NP = _G * _G
_N = _B * _NP
_ND = _C * _P * _P
_APAD = 256
_NCOPY = _C * _P
_RBLK = 448
_NSTEP = _N // _RBLK


@functools.lru_cache(maxsize=1)
def _static_graph():
    r, c = np.triu_indices(_NP, k=1)
    e = np.stack([r.astype(np.int64), c.astype(np.int64)])
    e = np.tile(e[None], (_B, 1, 1)).reshape(-1, 2).T
    row, col = e[0], e[1]
    deg = np.zeros((_N,), np.float64)
    np.add.at(deg, col, 1.0)
    deg += 1.0
    dinv = deg ** -0.5
    A = np.zeros((_APAD, _APAD), np.float64)
    np.add.at(A, (col, row), dinv[row] * dinv[col])
    idx = np.arange(_APAD)
    A[idx, idx] += dinv[:_APAD] ** 2
    Sp = np.zeros((_B, _N), np.float64)
    Sp[0, :_APAD] = A[:_NP, :].sum(axis=0) / _NP
    for g in range(1, _B):
        Sp[g, g * _NP:(g + 1) * _NP] = 1.0 / _NP
    return A.astype(np.float32), Sp.astype(np.float32)


def _prep_body(wemb_ref, w1_ref, w2_ref, w3_ref,
               wembT_ref, w1T_ref, w2T_ref, w3T_ref):
    wembT_ref[...] = wemb_ref[...].T
    w1T_ref[...] = w1_ref[...].T
    w2T_ref[...] = w2_ref[...].T
    w3T_ref[...] = w3_ref[...].T


def _main_body(pat_ref, wemb_ref, bemb_ref, w1_ref, b1_ref, w2_ref, b2_ref,
               w3_ref, b3_ref, a_ref, sp_ref, out_ref, xh_ref, t_ref):
    i = pl.program_id(0)
    xh_ref[pl.ds(i * _RBLK, _RBLK), :] = (
        jnp.dot(pat_ref[...], wemb_ref[...],
                preferred_element_type=jnp.float32) + bemb_ref[...])

    @pl.when(i == _NSTEP - 1)
    def _tail():
        a = a_ref[...]
        h = xh_ref[...]
        for w_ref, b_ref in ((w1_ref, b1_ref), (w2_ref, b2_ref)):
            t = jnp.dot(h, w_ref[...], preferred_element_type=jnp.float32)
            t_ref[...] = t
            t_ref[0:_APAD, :] = jnp.dot(a, t[0:_APAD, :],
                                        preferred_element_type=jnp.float32)
            h = jnp.maximum(t_ref[...] + b_ref[...], 0.0)
        p = jnp.dot(sp_ref[...], h, preferred_element_type=jnp.float32)
        out_ref[...] = (jnp.dot(p, w3_ref[...],
                                preferred_element_type=jnp.float32)
                        + b3_ref[...])


def kernel(x, W_embed, b_embed, W1, b1, W2, b2, W3, b3):
    A, Sp = _static_graph()
    pat = (x.reshape(_B, _C, _G, _P, _G, _P)
           .transpose(0, 2, 4, 1, 3, 5)
           .reshape(_N, _ND))
    vmem = pl.BlockSpec(memory_space=pltpu.MemorySpace.VMEM)
    f32 = jnp.float32
    wembT, w1T, w2T, w3T = pl.pallas_call(
        _prep_body,
        in_specs=[vmem, vmem, vmem, vmem],
        out_specs=(vmem, vmem, vmem, vmem),
        out_shape=(
            jax.ShapeDtypeStruct((_ND, _HID), f32),
            jax.ShapeDtypeStruct((_HID, _HID), f32),
            jax.ShapeDtypeStruct((_HID, _HID), f32),
            jax.ShapeDtypeStruct((_HID, _HID), f32),
        ),
    )(W_embed, W1, W2, W3)

    full = lambda shape: pl.BlockSpec(shape, lambda i: (0,) * len(shape))
    return pl.pallas_call(
        _main_body,
        grid=(_NSTEP,),
        in_specs=[
            pl.BlockSpec((_RBLK, _ND), lambda i: (i, 0)),
            full((_ND, _HID)),
            full((1, _HID)),
            full((_HID, _HID)),
            full((1, _HID)),
            full((_HID, _HID)),
            full((1, _HID)),
            full((_HID, _HID)),
            full((1, _HID)),
            full((_APAD, _APAD)),
            full((_B, _N)),
        ],
        out_specs=pl.BlockSpec((_B, _HID), lambda i: (0, 0)),
        out_shape=jax.ShapeDtypeStruct((_B, _HID), f32),
        scratch_shapes=[
            pltpu.MemorySpace.VMEM((_N, _HID), f32),
            pltpu.MemorySpace.VMEM((_N, _HID), f32),
        ],
    )(pat, wembT, b_embed.reshape(1, -1), w1T,
      b1.reshape(1, -1), w2T, b2.reshape(1, -1), w3T, b3.reshape(1, -1),
      jnp.asarray(A), jnp.asarray(Sp))


# speedup vs baseline: 1.0902x; 1.0074x over previous
import functools

import numpy as np
import jax
import jax.numpy as jnp
from jax import lax
from jax.experimental import pallas as pl
from jax.experimental.pallas import tpu as pltpu

_B, _C, _IMG, _P = 32, 3, 224, 16
_HID = 128
_G = _IMG // _P
_NP = _G * _G
_N = _B * _NP
_ND = _C * _P * _P
_APAD = 256
_BB = 2
_BIMG = _B // _BB
_BROWS = _BIMG * _NP
_KSTEP = _ND // 128


@functools.lru_cache(maxsize=1)
def _static_graph():
    r, c = np.triu_indices(_NP, k=1)
    e = np.stack([r.astype(np.int64), c.astype(np.int64)])
    e = np.tile(e[None], (_B, 1, 1)).reshape(-1, 2).T
    row, col = e[0], e[1]
    deg = np.zeros((_N,), np.float64)
    np.add.at(deg, col, 1.0)
    deg += 1.0
    dinv = deg ** -0.5
    A = np.zeros((_APAD, _APAD), np.float64)
    np.add.at(A, (col, row), dinv[row] * dinv[col])
    idx = np.arange(_APAD)
    A[idx, idx] += dinv[:_APAD] ** 2
    Sp = np.zeros((_B, _N), np.float64)
    Sp[0, :_APAD] = A[:_NP, :].sum(axis=0) / _NP
    for g in range(1, _B):
        Sp[g, g * _NP:(g + 1) * _NP] = 1.0 / _NP
    return A.astype(np.float32), Sp.astype(np.float32)


def _tmul(x, w):
    return lax.dot_general(x, w, (((1,), (1,)), ((), ())),
                           preferred_element_type=jnp.float32)


def _body(x_ref, wemb_ref, bemb_ref, w1_ref, b1_ref, w2_ref, b2_ref,
          w3_ref, b3_ref, a_ref, sp_ref, out_ref, xh_ref, t_ref, wembT_ref):
    b2 = pl.program_id(0)
    g = pl.program_id(1)

    @pl.when((b2 == 0) & (g == 0))
    def _once():
        wembT_ref[...] = wemb_ref[...].T

    v = x_ref[...].reshape(_BIMG, _G, 8, _G, _P)
    v128 = jnp.concatenate([v[:, :, i, :, :] for i in range(8)], axis=-1)
    v128 = v128.reshape(_BROWS, 128)
    wslice = wembT_ref[pl.ds(g * 128, 128), :]
    part = jnp.dot(v128, wslice, preferred_element_type=jnp.float32)
    rows = pl.ds(b2 * _BROWS, _BROWS)

    @pl.when(g == 0)
    def _init():
        xh_ref[rows, :] = part

    @pl.when(g > 0)
    def _acc():
        xh_ref[rows, :] += part

    @pl.when((b2 == _BB - 1) & (g == _KSTEP - 1))
    def _tail():
        a = a_ref[...]
        h = xh_ref[...] + bemb_ref[...]
        for w_ref, b_ref in ((w1_ref, b1_ref), (w2_ref, b2_ref)):
            t = _tmul(h, w_ref[...])
            t_ref[...] = t
            t_ref[0:_APAD, :] = jnp.dot(a, t[0:_APAD, :],
                                        preferred_element_type=jnp.float32)
            h = jnp.maximum(t_ref[...] + b_ref[...], 0.0)
        p = jnp.dot(sp_ref[...], h, preferred_element_type=jnp.float32)
        out_ref[...] = _tmul(p, w3_ref[...]) + b3_ref[...]


def kernel(x, W_embed, b_embed, W1, b1, W2, b2, W3, b3):
    A, Sp = _static_graph()
    x6 = x.reshape(_B, _C, _G, _P, _G, _P)
    full = lambda shape: pl.BlockSpec(shape, lambda b2, g: (0,) * len(shape))
    return pl.pallas_call(
        _body,
        grid=(_BB, _KSTEP),
        in_specs=[
            pl.BlockSpec((_BIMG, 1, _G, 8, _G, _P),
                         lambda b2, g: (b2, g // 2, 0, g % 2, 0, 0)),
            full((_HID, _ND)),
            full((1, _HID)),
            full((_HID, _HID)),
            full((1, _HID)),
            full((_HID, _HID)),
            full((1, _HID)),
            full((_HID, _HID)),
            full((1, _HID)),
            full((_APAD, _APAD)),
            full((_B, _N)),
        ],
        out_specs=pl.BlockSpec((_B, _HID), lambda b2, g: (0, 0)),
        out_shape=jax.ShapeDtypeStruct((_B, _HID), jnp.float32),
        scratch_shapes=[
            pltpu.MemorySpace.VMEM((_N, _HID), jnp.float32),
            pltpu.MemorySpace.VMEM((_N, _HID), jnp.float32),
            pltpu.MemorySpace.VMEM((_ND, _HID), jnp.float32),
        ],
    )(x6, W_embed, b_embed.reshape(1, -1), W1, b1.reshape(1, -1),
      W2, b2.reshape(1, -1), W3, b3.reshape(1, -1),
      jnp.asarray(A), jnp.asarray(Sp))


# speedup vs baseline: 4.6138x; 4.2319x over previous
import functools

import numpy as np
import jax
import jax.numpy as jnp
from jax import lax
from jax.experimental import pallas as pl
from jax.experimental.pallas import tpu as pltpu

_B, _C, _IMG, _P = 32, 3, 224, 16
_HID = 128
_G = _IMG // _P
_NP = _G * _G
_N = _B * _NP
_ND = _C * _P * _P
_APAD = 256
_BB = 2
_BIMG = _B // _BB
_BROWS = _BIMG * _NP


@functools.lru_cache(maxsize=1)
def _static_graph():
    r, c = np.triu_indices(_NP, k=1)
    e = np.stack([r.astype(np.int64), c.astype(np.int64)])
    e = np.tile(e[None], (_B, 1, 1)).reshape(-1, 2).T
    row, col = e[0], e[1]
    deg = np.zeros((_N,), np.float64)
    np.add.at(deg, col, 1.0)
    deg += 1.0
    dinv = deg ** -0.5
    A = np.zeros((_APAD, _APAD), np.float64)
    np.add.at(A, (col, row), dinv[row] * dinv[col])
    idx = np.arange(_APAD)
    A[idx, idx] += dinv[:_APAD] ** 2
    Sp = np.zeros((_B, _N), np.float64)
    Sp[0, :_APAD] = A[:_NP, :].sum(axis=0) / _NP
    for g in range(1, _B):
        Sp[g, g * _NP:(g + 1) * _NP] = 1.0 / _NP
    return A.astype(np.float32), Sp.astype(np.float32)


def _tmul(x, w):
    return lax.dot_general(x, w, (((1,), (1,)), ((), ())),
                           preferred_element_type=jnp.float32)


def _body(x_ref, wemb_ref, bemb_ref, w1_ref, b1_ref, w2_ref, b2_ref,
          w3_ref, b3_ref, a_ref, sp_ref, out_ref, xh_ref, t_ref, wembT_ref):
    b2 = pl.program_id(0)
    c = pl.program_id(1)

    @pl.when((b2 == 0) & (c == 0) & (pl.program_id(2) == 0))
    def _once():
        wembT_ref[...] = wemb_ref[...].T.reshape(2 * _C, _HID, _HID)

    v = x_ref[...].reshape(_BIMG, _G, 8, _IMG)
    ihi = pl.program_id(2)
    rows = pl.ds(b2 * _BROWS, _BROWS)
    vi = [v[:, :, il, :] for il in range(8)]
    blocks = []
    for px in range(_G):
        blocks.append(jnp.concatenate(
            [s[:, :, _P * px:_P * (px + 1)] for s in vi], axis=-1))
    v128 = jnp.stack(blocks, axis=2).reshape(_BROWS, _HID)
    w128 = wembT_ref[c * 2 + ihi]
    part = jnp.dot(v128, w128, preferred_element_type=jnp.float32)

    kstep = c * 2 + ihi

    @pl.when(kstep == 0)
    def _init():
        xh_ref[rows, :] = part

    @pl.when(kstep > 0)
    def _acc():
        xh_ref[rows, :] += part

    @pl.when((b2 == _BB - 1) & (kstep == 2 * _C - 1))
    def _tail():
        a = a_ref[...]
        h = xh_ref[...] + bemb_ref[...]
        for w_ref, b_ref in ((w1_ref, b1_ref), (w2_ref, b2_ref)):
            t = _tmul(h, w_ref[...])
            t_ref[...] = t
            t_ref[0:_APAD, :] = jnp.dot(a, t[0:_APAD, :],
                                        preferred_element_type=jnp.float32)
            h = jnp.maximum(t_ref[...] + b_ref[...], 0.0)
        p = jnp.dot(sp_ref[...], h, preferred_element_type=jnp.float32)
        out_ref[...] = _tmul(p, w3_ref[...]) + b3_ref[...]


def kernel(x, W_embed, b_embed, W1, b1, W2, b2, W3, b3):
    A, Sp = _static_graph()
    x5 = x.reshape(_B, _C, _G, _P, _IMG)
    full = lambda shape: pl.BlockSpec(
        shape, lambda b2, c, ihi: (0,) * len(shape))
    return pl.pallas_call(
        _body,
        grid=(_BB, _C, 2),
        in_specs=[
            pl.BlockSpec((_BIMG, 1, _G, 8, _IMG),
                         lambda b2, c, ihi: (b2, c, 0, ihi, 0)),
            full((_HID, _ND)),
            full((1, _HID)),
            full((_HID, _HID)),
            full((1, _HID)),
            full((_HID, _HID)),
            full((1, _HID)),
            full((_HID, _HID)),
            full((1, _HID)),
            full((_APAD, _APAD)),
            full((_B, _N)),
        ],
        out_specs=pl.BlockSpec((_B, _HID), lambda b2, c, ihi: (0, 0)),
        out_shape=jax.ShapeDtypeStruct((_B, _HID), jnp.float32),
        scratch_shapes=[
            pltpu.MemorySpace.VMEM((_N, _HID), jnp.float32),
            pltpu.MemorySpace.VMEM((_N, _HID), jnp.float32),
            pltpu.MemorySpace.VMEM((2 * _C, _HID, _HID), jnp.float32),
        ],
    )(x5, W_embed, b_embed.reshape(1, -1), W1, b1.reshape(1, -1),
      W2, b2.reshape(1, -1), W3, b3.reshape(1, -1),
      jnp.asarray(A), jnp.asarray(Sp))


# speedup vs baseline: 4.9398x; 1.0707x over previous
import functools

import numpy as np
import jax
import jax.numpy as jnp
from jax import lax
from jax.experimental import pallas as pl
from jax.experimental.pallas import tpu as pltpu

_B, _C, _IMG, _P = 32, 3, 224, 16
_HID = 128
_G = _IMG // _P
_NP = _G * _G
_N = _B * _NP
_ND = _C * _P * _P
_APAD = 256
_BB = 2
_BIMG = _B // _BB
_BROWS = _BIMG * _NP


@functools.lru_cache(maxsize=1)
def _static_graph():
    r, c = np.triu_indices(_NP, k=1)
    e = np.stack([r.astype(np.int64), c.astype(np.int64)])
    e = np.tile(e[None], (_B, 1, 1)).reshape(-1, 2).T
    row, col = e[0], e[1]
    deg = np.zeros((_N,), np.float64)
    np.add.at(deg, col, 1.0)
    deg += 1.0
    dinv = deg ** -0.5
    A = np.zeros((_APAD, _APAD), np.float64)
    np.add.at(A, (col, row), dinv[row] * dinv[col])
    idx = np.arange(_APAD)
    A[idx, idx] += dinv[:_APAD] ** 2
    Sp = np.zeros((_B, _N), np.float64)
    Sp[0, :_APAD] = A[:_NP, :].sum(axis=0) / _NP
    for g in range(1, _B):
        Sp[g, g * _NP:(g + 1) * _NP] = 1.0 / _NP
    return A.astype(np.float32), Sp.astype(np.float32)


def _tmul(x, w):
    return lax.dot_general(x, w, (((1,), (1,)), ((), ())),
                           preferred_element_type=jnp.float32)


def _body(x_ref, wemb_ref, bemb_ref, w1_ref, b1_ref, w2_ref, b2_ref,
          w3_ref, b3_ref, a_ref, sp_ref, out_ref, xh_ref, t_ref, wembT_ref):
    b2 = pl.program_id(0)
    c = pl.program_id(1)

    @pl.when((b2 == 0) & (c == 0) & (pl.program_id(2) == 0))
    def _once():
        wembT_ref[...] = wemb_ref[...].T.reshape(
            2 * _C, _HID, _HID).astype(jnp.bfloat16)

    v = x_ref[...].reshape(_BIMG, _G, 8, _IMG).astype(jnp.bfloat16)
    ihi = pl.program_id(2)
    rows = pl.ds(b2 * _BROWS, _BROWS)
    vi = [v[:, :, il, :] for il in range(8)]
    blocks = []
    for px in range(_G):
        blocks.append(jnp.concatenate(
            [s[:, :, _P * px:_P * (px + 1)] for s in vi], axis=-1))
    v128 = jnp.stack(blocks, axis=2).reshape(_BROWS, _HID)
    w128 = wembT_ref[c * 2 + ihi]
    part = jnp.dot(v128, w128, preferred_element_type=jnp.float32)

    kstep = c * 2 + ihi

    @pl.when(kstep == 0)
    def _init():
        xh_ref[rows, :] = part

    @pl.when(kstep > 0)
    def _acc():
        xh_ref[rows, :] += part

    @pl.when((b2 == _BB - 1) & (kstep == 2 * _C - 1))
    def _tail():
        a = a_ref[...]
        h = xh_ref[...] + bemb_ref[...]
        for w_ref, b_ref in ((w1_ref, b1_ref), (w2_ref, b2_ref)):
            t = _tmul(h, w_ref[...])
            t_ref[...] = t
            t_ref[0:_APAD, :] = jnp.dot(a, t[0:_APAD, :],
                                        preferred_element_type=jnp.float32)
            h = jnp.maximum(t_ref[...] + b_ref[...], 0.0)
        p = jnp.dot(sp_ref[...], h, preferred_element_type=jnp.float32)
        out_ref[...] = _tmul(p, w3_ref[...]) + b3_ref[...]


def kernel(x, W_embed, b_embed, W1, b1, W2, b2, W3, b3):
    A, Sp = _static_graph()
    x5 = x.reshape(_B, _C, _G, _P, _IMG)
    full = lambda shape: pl.BlockSpec(
        shape, lambda b2, c, ihi: (0,) * len(shape))
    return pl.pallas_call(
        _body,
        grid=(_BB, _C, 2),
        in_specs=[
            pl.BlockSpec((_BIMG, 1, _G, 8, _IMG),
                         lambda b2, c, ihi: (b2, c, 0, ihi, 0)),
            full((_HID, _ND)),
            full((1, _HID)),
            full((_HID, _HID)),
            full((1, _HID)),
            full((_HID, _HID)),
            full((1, _HID)),
            full((_HID, _HID)),
            full((1, _HID)),
            full((_APAD, _APAD)),
            full((_B, _N)),
        ],
        out_specs=pl.BlockSpec((_B, _HID), lambda b2, c, ihi: (0, 0)),
        out_shape=jax.ShapeDtypeStruct((_B, _HID), jnp.float32),
        scratch_shapes=[
            pltpu.MemorySpace.VMEM((_N, _HID), jnp.float32),
            pltpu.MemorySpace.VMEM((_N, _HID), jnp.float32),
            pltpu.MemorySpace.VMEM((2 * _C, _HID, _HID), jnp.bfloat16),
        ],
    )(x5, W_embed, b_embed.reshape(1, -1), W1, b1.reshape(1, -1),
      W2, b2.reshape(1, -1), W3, b3.reshape(1, -1),
      jnp.asarray(A), jnp.asarray(Sp))


# speedup vs baseline: 7.1661x; 1.4507x over previous
import functools

import numpy as np
import jax
import jax.numpy as jnp
from jax import lax
from jax.experimental import pallas as pl
from jax.experimental.pallas import tpu as pltpu

_B, _C, _IMG, _P = 32, 3, 224, 16
_HID = 128
_G = _IMG // _P
_NP = _G * _G
_N = _B * _NP
_ND = _C * _P * _P
_APAD = 256
_BB = 2
_BIMG = _B // _BB
_BROWS = _BIMG * _NP


@functools.lru_cache(maxsize=1)
def _static_graph():
    r, c = np.triu_indices(_NP, k=1)
    e = np.stack([r.astype(np.int64), c.astype(np.int64)])
    e = np.tile(e[None], (_B, 1, 1)).reshape(-1, 2).T
    row, col = e[0], e[1]
    deg = np.zeros((_N,), np.float64)
    np.add.at(deg, col, 1.0)
    deg += 1.0
    dinv = deg ** -0.5
    A = np.zeros((_APAD, _APAD), np.float64)
    np.add.at(A, (col, row), dinv[row] * dinv[col])
    idx = np.arange(_APAD)
    A[idx, idx] += dinv[:_APAD] ** 2
    o = (np.arange(_NP) % _G) * _G + np.arange(_NP) // _G
    opad = np.concatenate([o, np.arange(_NP, _APAD)])
    A = A[np.ix_(opad, opad)]
    Sp = np.zeros((_B, _N), np.float64)
    Sp[0, :_APAD] = A[:_NP, :].sum(axis=0) / _NP
    for g in range(1, _B):
        Sp[g, g * _NP:(g + 1) * _NP] = 1.0 / _NP
    return A.astype(np.float32), Sp.astype(np.float32)


def _tmul(x, w):
    return lax.dot_general(x, w, (((1,), (1,)), ((), ())),
                           preferred_element_type=jnp.float32)


def _body(x_ref, wemb_ref, bemb_ref, w1_ref, b1_ref, w2_ref, b2_ref,
          w3_ref, b3_ref, a_ref, sp_ref, out_ref, xh_ref, t_ref, wembT_ref):
    b2 = pl.program_id(0)
    c = pl.program_id(1)

    @pl.when((b2 == 0) & (c == 0) & (pl.program_id(2) == 0))
    def _once():
        wembT_ref[...] = wemb_ref[...].T.reshape(
            2 * _C, _HID, _HID).astype(jnp.bfloat16)

    v = x_ref[...].reshape(_BIMG, _G, 8, _IMG).astype(jnp.bfloat16)
    ihi = pl.program_id(2)
    rows = pl.ds(b2 * _BROWS, _BROWS)
    blocks = []
    for px in range(_G):
        blocks.append(jnp.concatenate(
            [v[:, :, il, _P * px:_P * (px + 1)] for il in range(8)], axis=-1))
    v128 = jnp.concatenate(blocks, axis=1).reshape(_BROWS, _HID)
    w128 = wembT_ref[c * 2 + ihi]
    part = jnp.dot(v128, w128, preferred_element_type=jnp.float32)

    kstep = c * 2 + ihi

    @pl.when(kstep == 0)
    def _init():
        xh_ref[rows, :] = part

    @pl.when(kstep > 0)
    def _acc():
        xh_ref[rows, :] += part

    @pl.when((b2 == _BB - 1) & (kstep == 2 * _C - 1))
    def _tail():
        a = a_ref[...]
        h = xh_ref[...] + bemb_ref[...]
        for w_ref, b_ref in ((w1_ref, b1_ref), (w2_ref, b2_ref)):
            t = _tmul(h, w_ref[...])
            t_ref[...] = t
            t_ref[0:_APAD, :] = jnp.dot(a, t[0:_APAD, :],
                                        preferred_element_type=jnp.float32)
            h = jnp.maximum(t_ref[...] + b_ref[...], 0.0)
        p = jnp.dot(sp_ref[...], h, preferred_element_type=jnp.float32)
        out_ref[...] = _tmul(p, w3_ref[...]) + b3_ref[...]


def kernel(x, W_embed, b_embed, W1, b1, W2, b2, W3, b3):
    A, Sp = _static_graph()
    x5 = x.reshape(_B, _C, _G, _P, _IMG)
    full = lambda shape: pl.BlockSpec(
        shape, lambda b2, c, ihi: (0,) * len(shape))
    return pl.pallas_call(
        _body,
        grid=(_BB, _C, 2),
        in_specs=[
            pl.BlockSpec((_BIMG, 1, _G, 8, _IMG),
                         lambda b2, c, ihi: (b2, c, 0, ihi, 0)),
            full((_HID, _ND)),
            full((1, _HID)),
            full((_HID, _HID)),
            full((1, _HID)),
            full((_HID, _HID)),
            full((1, _HID)),
            full((_HID, _HID)),
            full((1, _HID)),
            full((_APAD, _APAD)),
            full((_B, _N)),
        ],
        out_specs=pl.BlockSpec((_B, _HID), lambda b2, c, ihi: (0, 0)),
        out_shape=jax.ShapeDtypeStruct((_B, _HID), jnp.float32),
        scratch_shapes=[
            pltpu.MemorySpace.VMEM((_N, _HID), jnp.float32),
            pltpu.MemorySpace.VMEM((_N, _HID), jnp.float32),
            pltpu.MemorySpace.VMEM((2 * _C, _HID, _HID), jnp.bfloat16),
        ],
    )(x5, W_embed, b_embed.reshape(1, -1), W1, b1.reshape(1, -1),
      W2, b2.reshape(1, -1), W3, b3.reshape(1, -1),
      jnp.asarray(A), jnp.asarray(Sp))
